# baseline (speedup 1.0000x reference)
import jax
import jax.numpy as jnp
from jax import lax
from jax.experimental import pallas as pl
from jax.experimental.pallas import tpu as pltpu
from jax.experimental.pallas import tpu_sc as plsc

NEG = 0.01
DRN2 = 0.4 * 0.4
SENT = 1e9
B, P = 64, 128
NCHUNK = P // 16
NC, NS = 2, 16
JETS_PER_W = B // (NC * NS)

SIN_C = (6.28318503, -41.34161603, 81.60091368, -76.62655312, 41.4034446,
         -12.57638988)
COS_C = (0.99999999, -19.73920555, 64.93917218, -85.45116482, 60.1762218,
         -26.00049347, 6.57556005)

OFF0 = dict(Wf=0, Ws=5, Wd=10, wc=15, ws=20, b1=25, W2=30, b2=55, W3=60,
            b3=85)
OFF1 = dict(Wf=90, Ws=110, Wd=130, wc=150, ws=155, b1=160, W2=165, b2=190,
            W3=195, b3=210)
NW = 224
PK = plsc.PackFormat.INTERLEAVED


def _leaky(x):
    return jnp.where(x >= 0, x, NEG * x)


def _rinv(r2):
    i = lax.bitcast_convert_type(r2, jnp.int32)
    i = jnp.int32(0x5F3759DF) - (i >> 1)
    y = lax.bitcast_convert_type(i, jnp.float32)
    for _ in range(3):
        y = y * (1.5 - 0.5 * r2 * y * y)
    return y


def _rbf(x):
    i = lax.bitcast_convert_type(x, jnp.int32)
    i = (i + 0x7FFF + ((i >> 16) & 1)) & jnp.int32(-65536)
    return lax.bitcast_convert_type(i, jnp.float32)


def _rbf_fast(x):
    i = lax.bitcast_convert_type(x, jnp.int32)
    i = (i + 0x8000) & jnp.int32(-65536)
    return lax.bitcast_convert_type(i, jnp.float32)


def _roundn(vals):
    return [_rbf_fast(v) for v in vals]


def _sincos_2pi(phase):
    n = (phase + 0.5 * jnp.sign(phase)).astype(jnp.int32).astype(jnp.float32)
    t = phase - n
    z = t * t
    s = jnp.float32(SIN_C[5])
    for c in SIN_C[4::-1]:
        s = s * z + c
    s = s * t
    co = jnp.float32(COS_C[6])
    for c in COS_C[5::-1]:
        co = co * z + c
    return co, s


def _sc_body(pt_h, ax_h, ay_h, w_h, out_h,
             ptv, axv, ayv, axs, ays, uxv, uyv, znz, vfv,
             a0, a1, a2, a3, a4, b0, b1_, b2_, b3_, b4,
             m0, m1, m2, m3, f0v, wv, orow):
    wid = lax.axis_index("s") * NC + lax.axis_index("c")
    pltpu.sync_copy(w_h, wv)
    avs = (a0, a1, a2, a3, a4)
    bvs = (b0, b1_, b2_, b3_, b4)

    def wext():
        wch = [wv[pl.ds(i * 16, 16)] for i in range(NW // 16)]
        return lambda o: wch[o // 16][o % 16]

    def conv_pass(n_msg, nk, off, readout):
        fr = (f0v,) if nk == 1 else (m0, m1, m2, m3)
        wsc = wext()
        wd = [[wsc(off["Wd"] + k * 5 + c) for c in range(5)]
              for k in range(nk)]
        wc = [wsc(off["wc"] + c) for c in range(5)]
        ws = [wsc(off["ws"] + c) for c in range(5)]
        w2 = [[wsc(off["W2"] + k * 5 + c) for c in range(5)]
              for k in range(5)]
        bb2 = [wsc(off["b2"] + c) for c in range(5)]
        w3 = [[wsc(off["W3"] + k * n_msg + c) for c in range(n_msg)]
              for k in range(5)]
        bb3 = [wsc(off["b3"] + c) for c in range(n_msg)]

        def ibody(ic, scarry):
            base = pl.multiple_of(ic * 16, 8)
            sl = pl.ds(base, 16)
            axi = axs[sl]
            ayi = ays[sl]
            uxi = uxv[sl]
            uyi = uyv[sl]
            zi = znz[sl]
            fi = [fr[k][sl] for k in range(nk)]
            ai = [avs[c][sl] for c in range(5)]

            def jbody(j, carry):
                jf = jnp.full((16,), j, jnp.int32)
                axj = plsc.load_gather(axs, [jf])
                ayj = plsc.load_gather(ays, [jf])
                dx = axi - axj
                dy = ayi - ayj
                d2 = dx * dx + dy * dy
                mask = d2 <= DRN2

                def hit(carry):
                    accs, deg = carry[:-1], carry[-1]
                    uxj = plsc.load_gather(uxv, [jf])
                    uyj = plsc.load_gather(uyv, [jf])
                    zj = plsc.load_gather(znz, [jf])
                    bj = [plsc.load_gather(bvs[c], [jf]) for c in range(5)]
                    fj = [plsc.load_gather(fr[k], [jf]) for k in range(nk)]
                    adjf = jnp.where(mask, 1.0, 0.0).astype(jnp.float32)
                    aw = adjf * (zi * zj)
                    cs = uxi * uxj + uyi * uyj
                    sn = uyi * uxj - uxi * uyj
                    rvals = _roundn([fj[k] - fi[k] for k in range(nk)]
                                    + [cs, sn])
                    dlt, csr, snr = rvals[:nk], rvals[nk], rvals[nk + 1]
                    h = []
                    for c in range(5):
                        t = ai[c] + bj[c] + csr * wc[c] + snr * ws[c]
                        for k in range(nk):
                            t = t + dlt[k] * wd[k][c]
                        h.append(_leaky(t))
                    h = _roundn(h)
                    h2 = []
                    for c in range(5):
                        t = h[0] * w2[0][c] + bb2[c]
                        for k in range(1, 5):
                            t = t + h[k] * w2[k][c]
                        h2.append(_leaky(t))
                    h2 = _roundn(h2)
                    out = []
                    for c in range(n_msg):
                        t = h2[0] * w3[0][c] + bb3[c]
                        for k in range(1, 5):
                            t = t + h2[k] * w3[k][c]
                        out.append(t)
                    accs = tuple(accs[c] + out[c] * aw
                                 for c in range(n_msg))
                    return accs + (deg + adjf,)

                return lax.cond(jnp.any(mask), hit, lambda c: c, carry)

            z16 = jnp.zeros((16,), jnp.float32)
            init = tuple(z16 for _ in range(n_msg)) + (z16,)
            res = lax.fori_loop(0, P, jbody, init, unroll=4)
            accs, deg = res[:-1], res[-1]
            pti = ptv[sl]
            vf = vfv[sl]
            cx = axv[sl]
            cy = ayv[sl]
            denom = pti * deg
            dsafe = jnp.where(denom == 0, 1.0, denom)
            inv = _rinv(dsafe)
            inv = inv * inv
            scale = pti * inv * vf
            g = pti * deg * inv * vf
            res_m = [accs[c] * scale for c in range(n_msg)]
            phase = res_m[n_msg - 1]
            co, sn = _sincos_2pi(phase)
            gx = cx * g
            gy = cy * g
            rx = co * gx - sn * gy
            ry = sn * gx + co * gy
            pt_new = pti * g
            if not readout:
                ptv[sl] = pt_new
                m0[sl] = res_m[0]
                m1[sl] = res_m[1]
                m2[sl] = res_m[2]
                m3[sl] = res_m[3]
                axv[sl] = rx
                ayv[sl] = ry
                return scarry
            ptn = pt_new * vf
            return (scarry[0] + ptn,
                    scarry[1] + ptn * res_m[0],
                    scarry[2] + ptn * res_m[1],
                    scarry[3] + ptn * rx,
                    scarry[4] + ptn * ry)

        z16 = jnp.zeros((16,), jnp.float32)
        return lax.fori_loop(0, NCHUNK, ibody, (z16,) * 5)

    def stage(conv1, nk, off):
        wsc = wext()
        wf = [[wsc(off["Wf"] + k * 5 + c) for c in range(5)]
              for k in range(nk)]
        wsum = [[wsc(off["Ws"] + k * 5 + c) for c in range(5)]
                for k in range(nk)]
        bb1 = [wsc(off["b1"] + c) for c in range(5)]

        def sbody(ic, _):
            base = pl.multiple_of(ic * 16, 8)
            sl = pl.ds(base, 16)
            cx = axv[sl]
            cy = ayv[sl]
            vf = vfv[sl]
            r2 = cx * cx + cy * cy
            pos = r2 > 0
            ri = _rinv(r2)
            ux = jnp.where(pos, cx * ri, 0.0)
            uy = jnp.where(pos, cy * ri, 0.0)
            zz = jnp.where(pos, 1.0, 0.0).astype(jnp.float32)
            axs[sl] = jnp.where(vf > 0, cx, SENT)
            ays[sl] = jnp.where(vf > 0, cy, SENT)
            uxv[sl] = ux
            uyv[sl] = uy
            znz[sl] = zz
            if conv1:
                fk = [m0[sl], m1[sl], m2[sl], m3[sl]]
            else:
                rr = jnp.where(pos, r2 * ri, 0.0)
                f0v[sl] = rr
                fk = [rr]
            fb = _roundn(fk)
            for c in range(5):
                ta = fb[0] * wf[0][c]
                tb = fb[0] * wsum[0][c] + bb1[c]
                for k in range(1, nk):
                    ta = ta + fb[k] * wf[k][c]
                    tb = tb + fb[k] * wsum[k][c]
                avs[c][sl] = ta
                bvs[c][sl] = tb
            return 0

        lax.fori_loop(0, NCHUNK, sbody, 0)

    def jet(bl, _):
        b = wid * JETS_PER_W + bl
        pltpu.sync_copy(pt_h.at[b], ptv)
        pltpu.sync_copy(ax_h.at[b], axv)
        pltpu.sync_copy(ay_h.at[b], ayv)

        def vbody(ic, _):
            base = pl.multiple_of(ic * 16, 8)
            sl = pl.ds(base, 16)
            vfv[sl] = jnp.where(jnp.abs(axv[sl]) + jnp.abs(ayv[sl]) != 0,
                                1.0, 0.0).astype(jnp.float32)
            return 0

        lax.fori_loop(0, NCHUNK, vbody, 0)
        stage(False, 1, OFF0)
        conv_pass(5, 1, OFF0, readout=False)
        stage(True, 4, OFF1)
        s_ptn, s0, s1, s2, s3 = conv_pass(3, 4, OFF1, readout=True)
        den = jnp.sum(s_ptn)
        dsafe = jnp.where(den == 0, 1.0, den)
        dv = jnp.full((16,), dsafe, jnp.float32)
        iv = _rinv(dv)
        iv = iv * iv
        lane = lax.iota(jnp.int32, 16)
        outv = jnp.where(lane == 0, jnp.sum(s0),
                         jnp.where(lane == 1, jnp.sum(s1),
                                   jnp.where(lane == 2, jnp.sum(s2),
                                             jnp.where(lane == 3, jnp.sum(s3),
                                                       0.0))))
        orow[...] = outv.astype(jnp.float32) * iv
        pltpu.sync_copy(orow, out_h.at[b])
        return 0

    lax.fori_loop(0, JETS_PER_W, jet, 0)


@jax.jit
def _sc_conv(pt, ax, ay, wflat):
    mesh = plsc.VectorSubcoreMesh(core_axis_name="c", subcore_axis_name="s")
    f32 = jnp.float32
    scr = [pltpu.VMEM((P,), f32) for _ in range(24)] + [
        pltpu.VMEM((NW,), f32), pltpu.VMEM((16,), f32)]
    run = pl.kernel(_sc_body, mesh=mesh,
                    out_type=jax.ShapeDtypeStruct((B, 16), f32),
                    scratch_types=scr,
                    compiler_params=pltpu.CompilerParams(
                        needs_layout_passes=False))
    return run(pt, ax, ay, wflat)


def _readout_body(x_ref, w1_ref, b1_ref, w2_ref, b2_ref, w3_ref, b3_ref,
                  o_ref):
    def dot(x, w):
        return jnp.dot(_rbf(x), _rbf(w), precision=jax.lax.Precision.HIGHEST,
                       preferred_element_type=jnp.float32)

    x = x_ref[...]
    h = _leaky(dot(x, w1_ref[...]) + b1_ref[...][None, :])
    h = _leaky(dot(h, w2_ref[...]) + b2_ref[...][None, :])
    o_ref[...] = dot(h, w3_ref[...]) + b3_ref[...][None, :]


def kernel(pt, angles, mlp0, mlp1, mlp_readout):
    ax = angles[..., 0]
    ay = angles[..., 1]
    order = jnp.argsort(ax * ax + ay * ay, axis=1)
    pt = jnp.take_along_axis(pt, order, axis=1)
    ax = jnp.take_along_axis(ax, order, axis=1)
    ay = jnp.take_along_axis(ay, order, axis=1)

    bfr = _rbf

    def prep(params, k):
        (w1, bb1), (w2, bb2), (w3, bb3) = params
        return [bfr(w1[:k]).ravel(), bfr(w1[k:2 * k]).ravel(),
                bfr(w1[2 * k:3 * k]).ravel(), bfr(w1[3 * k]),
                bfr(w1[3 * k + 1]), bb1, bfr(w2).ravel(), bb2,
                bfr(w3).ravel(), bb3]

    wflat = jnp.concatenate(prep(mlp0, 1) + prep(mlp1, 4)
                            + [jnp.zeros((NW - 213,), jnp.float32)])
    out = _sc_conv(pt, ax, ay, wflat)
    agg = out[:, :4]
    (w1, b1), (w2, b2), (w3, b3) = mlp_readout
    ro = pl.pallas_call(
        _readout_body,
        out_shape=jax.ShapeDtypeStruct((B, 1), jnp.float32),
    )(agg[:, :2], w1, b1, w2, b2, w3, b3)
    return jnp.concatenate([ro, agg[:, 2:4]], axis=1)


# speedup vs baseline: 1.1532x; 1.1532x over previous
import jax
import jax.numpy as jnp
from jax import lax
from jax.experimental import pallas as pl
from jax.experimental.pallas import tpu as pltpu
from jax.experimental.pallas import tpu_sc as plsc

NEG = 0.01
DRN2 = 0.4 * 0.4
SENT = 1e9
B, P = 64, 128
NCHUNK = P // 16
NC, NS = 2, 16
JETS_PER_W = B // (NC * NS)

SIN_C = (6.28318503, -41.34161603, 81.60091368, -76.62655312, 41.4034446,
         -12.57638988)
COS_C = (0.99999999, -19.73920555, 64.93917218, -85.45116482, 60.1762218,
         -26.00049347, 6.57556005)

OFF0 = dict(Wf=0, Ws=5, Wd=10, wc=15, ws=20, b1=25, W2=30, b2=55, W3=60,
            b3=85)
OFF1 = dict(Wf=90, Ws=110, Wd=130, wc=150, ws=155, b1=160, W2=165, b2=190,
            W3=195, b3=210)
NW = 224
PK = plsc.PackFormat.INTERLEAVED


def _leaky(x):
    return jnp.where(x >= 0, x, NEG * x)


def _rinv(r2):
    i = lax.bitcast_convert_type(r2, jnp.int32)
    i = jnp.int32(0x5F3759DF) - (i >> 1)
    y = lax.bitcast_convert_type(i, jnp.float32)
    for _ in range(3):
        y = y * (1.5 - 0.5 * r2 * y * y)
    return y


def _rbf(x):
    i = lax.bitcast_convert_type(x, jnp.int32)
    i = (i + 0x7FFF + ((i >> 16) & 1)) & jnp.int32(-65536)
    return lax.bitcast_convert_type(i, jnp.float32)


def _rbf_fast(x):
    i = lax.bitcast_convert_type(x, jnp.int32)
    i = (i + 0x8000) & jnp.int32(-65536)
    return lax.bitcast_convert_type(i, jnp.float32)


def _roundn(vals):
    return [_rbf_fast(v) for v in vals]


def _sincos_2pi(phase):
    n = (phase + 0.5 * jnp.sign(phase)).astype(jnp.int32).astype(jnp.float32)
    t = phase - n
    z = t * t
    s = jnp.float32(SIN_C[5])
    for c in SIN_C[4::-1]:
        s = s * z + c
    s = s * t
    co = jnp.float32(COS_C[6])
    for c in COS_C[5::-1]:
        co = co * z + c
    return co, s


def _sc_body(pt_h, ax_h, ay_h, w_h, out_h,
             ptv, axv, ayv, axs, ays, uxv, uyv, znz, vfv,
             a0, a1, a2, a3, a4, b0, b1_, b2_, b3_, b4,
             m0, m1, m2, m3, f0v, wv, orow):
    wid = lax.axis_index("s") * NC + lax.axis_index("c")
    pltpu.sync_copy(w_h, wv)
    avs = (a0, a1, a2, a3, a4)
    bvs = (b0, b1_, b2_, b3_, b4)

    def wext():
        wch = [wv[pl.ds(i * 16, 16)] for i in range(NW // 16)]
        return lambda o: wch[o // 16][o % 16]

    def conv_pass(n_msg, nk, off, readout):
        fr = (f0v,) if nk == 1 else (m0, m1, m2, m3)
        wsc = wext()
        wd = [[wsc(off["Wd"] + k * 5 + c) for c in range(5)]
              for k in range(nk)]
        wc = [wsc(off["wc"] + c) for c in range(5)]
        ws = [wsc(off["ws"] + c) for c in range(5)]
        w2 = [[wsc(off["W2"] + k * 5 + c) for c in range(5)]
              for k in range(5)]
        bb2 = [wsc(off["b2"] + c) for c in range(5)]
        w3 = [[wsc(off["W3"] + k * n_msg + c) for c in range(n_msg)]
              for k in range(5)]
        bb3 = [wsc(off["b3"] + c) for c in range(n_msg)]

        def ibody(ic, scarry):
            base = pl.multiple_of(ic * 16, 8)
            sl = pl.ds(base, 16)
            axi = axs[sl]
            ayi = ays[sl]
            uxi = uxv[sl]
            uyi = uyv[sl]
            zi = znz[sl]
            fi = [fr[k][sl] for k in range(nk)]
            ai = [avs[c][sl] for c in range(5)]
            ri_ = f0v[sl]
            rlo = jnp.min(ri_) - (0.4 + 1e-4)
            rhi = jnp.max(ri_) + (0.4 + 1e-4)

            def jbody(j, carry):
                jf = jnp.full((16,), j, jnp.int32)
                axj = plsc.load_gather(axs, [jf])
                ayj = plsc.load_gather(ays, [jf])
                dx = axi - axj
                dy = ayi - ayj
                d2 = dx * dx + dy * dy
                mask = d2 <= DRN2

                def hit(carry):
                    accs, deg = carry[:-1], carry[-1]
                    uxj = plsc.load_gather(uxv, [jf])
                    uyj = plsc.load_gather(uyv, [jf])
                    zj = plsc.load_gather(znz, [jf])
                    bj = [plsc.load_gather(bvs[c], [jf]) for c in range(5)]
                    fj = [plsc.load_gather(fr[k], [jf]) for k in range(nk)]
                    adjf = jnp.where(mask, 1.0, 0.0).astype(jnp.float32)
                    aw = adjf * (zi * zj)
                    cs = uxi * uxj + uyi * uyj
                    sn = uyi * uxj - uxi * uyj
                    rvals = _roundn([fj[k] - fi[k] for k in range(nk)]
                                    + [cs, sn])
                    dlt, csr, snr = rvals[:nk], rvals[nk], rvals[nk + 1]
                    h = []
                    for c in range(5):
                        t = ai[c] + bj[c] + csr * wc[c] + snr * ws[c]
                        for k in range(nk):
                            t = t + dlt[k] * wd[k][c]
                        h.append(_leaky(t))
                    h = _roundn(h)
                    h2 = []
                    for c in range(5):
                        t = h[0] * w2[0][c] + bb2[c]
                        for k in range(1, 5):
                            t = t + h[k] * w2[k][c]
                        h2.append(_leaky(t))
                    h2 = _roundn(h2)
                    out = []
                    for c in range(n_msg):
                        t = h2[0] * w3[0][c] + bb3[c]
                        for k in range(1, 5):
                            t = t + h2[k] * w3[k][c]
                        out.append(t)
                    accs = tuple(accs[c] + out[c] * aw
                                 for c in range(n_msg))
                    return accs + (deg + adjf,)

                return lax.cond(jnp.any(mask), hit, lambda c: c, carry)

            def jcbody(jc, carry):
                jbase = pl.multiple_of(jc * 16, 8)
                rj = f0v[pl.ds(jbase, 16)]
                ok = jnp.any((rj >= rlo) & (rj <= rhi))
                return lax.cond(
                    ok,
                    lambda c: lax.fori_loop(jbase, jbase + 16, jbody, c),
                    lambda c: c, carry)

            z16 = jnp.zeros((16,), jnp.float32)
            init = tuple(z16 for _ in range(n_msg)) + (z16,)
            res = lax.fori_loop(0, NCHUNK, jcbody, init)
            accs, deg = res[:-1], res[-1]
            pti = ptv[sl]
            vf = vfv[sl]
            cx = axv[sl]
            cy = ayv[sl]
            denom = pti * deg
            dsafe = jnp.where(denom == 0, 1.0, denom)
            inv = _rinv(dsafe)
            inv = inv * inv
            scale = pti * inv * vf
            g = pti * deg * inv * vf
            res_m = [accs[c] * scale for c in range(n_msg)]
            phase = res_m[n_msg - 1]
            co, sn = _sincos_2pi(phase)
            gx = cx * g
            gy = cy * g
            rx = co * gx - sn * gy
            ry = sn * gx + co * gy
            pt_new = pti * g
            if not readout:
                ptv[sl] = pt_new
                m0[sl] = res_m[0]
                m1[sl] = res_m[1]
                m2[sl] = res_m[2]
                m3[sl] = res_m[3]
                axv[sl] = rx
                ayv[sl] = ry
                return scarry
            ptn = pt_new * vf
            return (scarry[0] + ptn,
                    scarry[1] + ptn * res_m[0],
                    scarry[2] + ptn * res_m[1],
                    scarry[3] + ptn * rx,
                    scarry[4] + ptn * ry)

        z16 = jnp.zeros((16,), jnp.float32)
        return lax.fori_loop(0, NCHUNK, ibody, (z16,) * 5)

    def stage(conv1, nk, off):
        wsc = wext()
        wf = [[wsc(off["Wf"] + k * 5 + c) for c in range(5)]
              for k in range(nk)]
        wsum = [[wsc(off["Ws"] + k * 5 + c) for c in range(5)]
                for k in range(nk)]
        bb1 = [wsc(off["b1"] + c) for c in range(5)]

        def sbody(ic, _):
            base = pl.multiple_of(ic * 16, 8)
            sl = pl.ds(base, 16)
            cx = axv[sl]
            cy = ayv[sl]
            vf = vfv[sl]
            r2 = cx * cx + cy * cy
            pos = r2 > 0
            ri = _rinv(r2)
            ux = jnp.where(pos, cx * ri, 0.0)
            uy = jnp.where(pos, cy * ri, 0.0)
            zz = jnp.where(pos, 1.0, 0.0).astype(jnp.float32)
            axs[sl] = jnp.where(vf > 0, cx, SENT)
            ays[sl] = jnp.where(vf > 0, cy, SENT)
            uxv[sl] = ux
            uyv[sl] = uy
            znz[sl] = zz
            if conv1:
                f0v[sl] = jnp.where(pos, r2 * ri, 0.0)
                fk = [m0[sl], m1[sl], m2[sl], m3[sl]]
            else:
                rr = jnp.where(pos, r2 * ri, 0.0)
                f0v[sl] = rr
                fk = [rr]
            fb = _roundn(fk)
            for c in range(5):
                ta = fb[0] * wf[0][c]
                tb = fb[0] * wsum[0][c] + bb1[c]
                for k in range(1, nk):
                    ta = ta + fb[k] * wf[k][c]
                    tb = tb + fb[k] * wsum[k][c]
                avs[c][sl] = ta
                bvs[c][sl] = tb
            return 0

        lax.fori_loop(0, NCHUNK, sbody, 0)

    def jet(bl, _):
        b = wid * JETS_PER_W + bl
        pltpu.sync_copy(pt_h.at[b], ptv)
        pltpu.sync_copy(ax_h.at[b], axv)
        pltpu.sync_copy(ay_h.at[b], ayv)

        def vbody(ic, _):
            base = pl.multiple_of(ic * 16, 8)
            sl = pl.ds(base, 16)
            vfv[sl] = jnp.where(jnp.abs(axv[sl]) + jnp.abs(ayv[sl]) != 0,
                                1.0, 0.0).astype(jnp.float32)
            return 0

        lax.fori_loop(0, NCHUNK, vbody, 0)
        stage(False, 1, OFF0)
        conv_pass(5, 1, OFF0, readout=False)
        stage(True, 4, OFF1)
        s_ptn, s0, s1, s2, s3 = conv_pass(3, 4, OFF1, readout=True)
        den = jnp.sum(s_ptn)
        dsafe = jnp.where(den == 0, 1.0, den)
        dv = jnp.full((16,), dsafe, jnp.float32)
        iv = _rinv(dv)
        iv = iv * iv
        lane = lax.iota(jnp.int32, 16)
        outv = jnp.where(lane == 0, jnp.sum(s0),
                         jnp.where(lane == 1, jnp.sum(s1),
                                   jnp.where(lane == 2, jnp.sum(s2),
                                             jnp.where(lane == 3, jnp.sum(s3),
                                                       0.0))))
        orow[...] = outv.astype(jnp.float32) * iv
        pltpu.sync_copy(orow, out_h.at[b])
        return 0

    lax.fori_loop(0, JETS_PER_W, jet, 0)


@jax.jit
def _sc_conv(pt, ax, ay, wflat):
    mesh = plsc.VectorSubcoreMesh(core_axis_name="c", subcore_axis_name="s")
    f32 = jnp.float32
    scr = [pltpu.VMEM((P,), f32) for _ in range(24)] + [
        pltpu.VMEM((NW,), f32), pltpu.VMEM((16,), f32)]
    run = pl.kernel(_sc_body, mesh=mesh,
                    out_type=jax.ShapeDtypeStruct((B, 16), f32),
                    scratch_types=scr,
                    compiler_params=pltpu.CompilerParams(
                        needs_layout_passes=False))
    return run(pt, ax, ay, wflat)


def _readout_body(x_ref, w1_ref, b1_ref, w2_ref, b2_ref, w3_ref, b3_ref,
                  o_ref):
    def dot(x, w):
        return jnp.dot(_rbf(x), _rbf(w), precision=jax.lax.Precision.HIGHEST,
                       preferred_element_type=jnp.float32)

    x = x_ref[...]
    h = _leaky(dot(x, w1_ref[...]) + b1_ref[...][None, :])
    h = _leaky(dot(h, w2_ref[...]) + b2_ref[...][None, :])
    o_ref[...] = dot(h, w3_ref[...]) + b3_ref[...][None, :]


def kernel(pt, angles, mlp0, mlp1, mlp_readout):
    ax = angles[..., 0]
    ay = angles[..., 1]
    order = jnp.argsort(ax * ax + ay * ay, axis=1)
    pt = jnp.take_along_axis(pt, order, axis=1)
    ax = jnp.take_along_axis(ax, order, axis=1)
    ay = jnp.take_along_axis(ay, order, axis=1)

    bfr = _rbf

    def prep(params, k):
        (w1, bb1), (w2, bb2), (w3, bb3) = params
        return [bfr(w1[:k]).ravel(), bfr(w1[k:2 * k]).ravel(),
                bfr(w1[2 * k:3 * k]).ravel(), bfr(w1[3 * k]),
                bfr(w1[3 * k + 1]), bb1, bfr(w2).ravel(), bb2,
                bfr(w3).ravel(), bb3]

    wflat = jnp.concatenate(prep(mlp0, 1) + prep(mlp1, 4)
                            + [jnp.zeros((NW - 213,), jnp.float32)])
    out = _sc_conv(pt, ax, ay, wflat)
    agg = out[:, :4]
    (w1, b1), (w2, b2), (w3, b3) = mlp_readout
    ro = pl.pallas_call(
        _readout_body,
        out_shape=jax.ShapeDtypeStruct((B, 1), jnp.float32),
    )(agg[:, :2], w1, b1, w2, b2, w3, b3)
    return jnp.concatenate([ro, agg[:, 2:4]], axis=1)
